# baseline (speedup 1.0000x reference)
import functools

import jax
import jax.numpy as jnp
from jax import lax
from jax.experimental import pallas as pl
from jax.experimental.pallas import tpu as pltpu
from jax.experimental.pallas import tpu_sc as plsc

N_NODES = 10000
NPAD = 10240
D_FEAT = 10000
H_DIM = 64
C_DIM = 16
E_EDGES = 320000
EPAD = 327680
SUB = 128
NSUB = EPAD // SUB
JUNK = N_NODES
K_HOPS = 10

NPAD4 = 4 * NPAD
EPAD4 = 4 * EPAD
NSUB4 = EPAD4 // SUB
JUNK4 = 4 * N_NODES

NC, NS = 2, 16
RT = NPAD // NS
RT4 = NPAD4 // NS
_MESH = plsc.VectorSubcoreMesh(core_axis_name="c", subcore_axis_name="s")


def _rsqrt_vec(x):
    i = plsc.bitcast(x, jnp.int32)
    i = jnp.int32(0x5F3759DF) - lax.shift_right_arithmetic(i, 1)
    y = plsc.bitcast(i, jnp.float32)
    for _ in range(3):
        y = y * (1.5 - 0.5 * x * y * y)
    return y


@functools.partial(
    pl.kernel,
    out_type=jax.ShapeDtypeStruct((NPAD, C_DIM), jnp.float32),
    mesh=_MESH,
    compiler_params=pltpu.CompilerParams(use_tc_tiling_on_sc=False, needs_layout_passes=False),
    scratch_types=(
        pltpu.VMEM_SHARED((NPAD4, C_DIM), jnp.float32),
        pltpu.VMEM_SHARED((NPAD, C_DIM), jnp.float32),
        pltpu.VMEM((16, SUB), jnp.int32),
        pltpu.VMEM((16, SUB), jnp.int32),
        pltpu.VMEM((16 * SUB, C_DIM), jnp.float32),
        pltpu.VMEM((H_DIM, C_DIM), jnp.float32),
        pltpu.VMEM((RT, C_DIM), jnp.float32),
        pltpu.VMEM((RT, C_DIM), jnp.float32),
        pltpu.VMEM((RT, C_DIM), jnp.float32),
        pltpu.VMEM((RT, C_DIM), jnp.float32),
        pltpu.SemaphoreType.DMA,
        pltpu.SemaphoreType.DMA,
    ),
)
def _appnp_sc(ar2, ac2, es2, ed2, w14, w2_hbm, zeros16, ones_hbm,
              xfin, x4_sh, y_sh, sbuf, dbuf, rows,
              w2buf, xbuf, h01b, d16b, zbuf, gsem, ssem):
    cid = lax.axis_index("c")
    sid = lax.axis_index("s")
    r4 = sid * RT4
    r0 = sid * RT

    def _drain(sem):
        pltpu.make_async_copy(zeros16.at[pl.ds(0, SUB)],
                              rows.at[pl.ds(0, SUB)], sem).wait()

    @pl.when(cid == 0)
    def _body():
        pltpu.sync_copy(zeros16.at[pl.ds(0, RT4)], x4_sh.at[pl.ds(r4, RT4)])
        pltpu.sync_copy(w2_hbm, w2buf)
        plsc.subcore_barrier()

        def sgrp(g, c2):
            p = lax.rem(g, 2) * 8
            gs = sid * (NSUB4 // NS) + g * 8
            pltpu.sync_copy(ac2.at[pl.ds(gs, 8)], sbuf.at[pl.ds(p, 8)])
            pltpu.sync_copy(ar2.at[pl.ds(gs, 8)], dbuf.at[pl.ds(p, 8)])
            for b in range(8):
                pltpu.async_copy(w14.at[sbuf.at[p + b]],
                                 rows.at[pl.ds((p + b) * SUB, SUB)], gsem)

            @pl.when(g > 0)
            def _dp():
                for b in range(8):
                    _drain(ssem)
            for b in range(8):
                _drain(gsem)
            for b in range(8):
                pltpu.async_copy(rows.at[pl.ds((p + b) * SUB, SUB)],
                                 x4_sh.at[dbuf.at[p + b]], ssem, add=True)
            return c2
        lax.fori_loop(0, NSUB4 // NS // 8, sgrp, 0)
        for b in range(8):
            _drain(ssem)

        plsc.subcore_barrier()

        def mm_chunk(ch, c2):
            pltpu.sync_copy(x4_sh.at[pl.ds(r4 + ch * 256, 256)],
                            rows.at[pl.ds(0, 256)])

            def mm_node(i, c3):
                node = ch * 64 + i
                acc = jnp.zeros((C_DIM,), jnp.float32)
                for q in range(4):
                    v = jnp.maximum(rows[4 * i + q], 0.0)
                    for j in range(C_DIM):
                        acc = acc + v[j] * w2buf[q * C_DIM + j]
                xbuf[node] = acc
                h01b[node] = 0.1 * acc
                return c3
            lax.fori_loop(0, 64, mm_node, 0)
            return c2
        lax.fori_loop(0, RT // 64, mm_chunk, 0)
        pltpu.sync_copy(zeros16.at[pl.ds(r0, RT)], x4_sh.at[pl.ds(r0, RT)])
        plsc.subcore_barrier()

        pltpu.sync_copy(ones_hbm, rows.at[pl.ds(0, 8 * SUB)])

        def dgrp(g, c2):
            gs = sid * (NSUB // NS) + g * 8
            pltpu.sync_copy(ed2.at[pl.ds(gs, 8)], sbuf.at[pl.ds(0, 8)])
            dls = [pltpu.async_copy(rows.at[pl.ds(b * SUB, SUB)],
                                    x4_sh.at[sbuf.at[b]], ssem, add=True)
                   for b in range(8)]
            for de in dls:
                de.wait()
            return c2
        lax.fori_loop(0, NSUB // NS // 8, dgrp, 0)
        plsc.subcore_barrier()

        pltpu.sync_copy(x4_sh.at[pl.ds(r0, RT)], zbuf)
        pltpu.sync_copy(zeros16.at[pl.ds(r0, RT)], x4_sh.at[pl.ds(r0, RT)])

        def norm(i, c2):
            dv = _rsqrt_vec(zbuf[i] + 1.0)
            d16b[i] = dv
            rows[1024 + i] = dv * xbuf[i]
            return c2
        lax.fori_loop(0, RT, norm, 0)
        pltpu.sync_copy(rows.at[pl.ds(1024, RT)], y_sh.at[pl.ds(r0, RT)])
        plsc.subcore_barrier()

        def hop(k, c):
            def grp(g, c2):
                p = lax.rem(g, 2) * 8
                gs = sid * (NSUB // NS) + g * 8
                pltpu.sync_copy(es2.at[pl.ds(gs, 8)], sbuf.at[pl.ds(p, 8)])
                pltpu.sync_copy(ed2.at[pl.ds(gs, 8)], dbuf.at[pl.ds(p, 8)])
                for b in range(8):
                    pltpu.async_copy(y_sh.at[sbuf.at[p + b]],
                                     rows.at[pl.ds((p + b) * SUB, SUB)], gsem)

                @pl.when(g > 0)
                def _dp():
                    for b in range(8):
                        _drain(ssem)
                for b in range(8):
                    _drain(gsem)
                for b in range(8):
                    pltpu.async_copy(rows.at[pl.ds((p + b) * SUB, SUB)],
                                     x4_sh.at[dbuf.at[p + b]], ssem, add=True)
                return c2
            lax.fori_loop(0, NSUB // NS // 8, grp, 0)
            for b in range(8):
                _drain(ssem)
            plsc.subcore_barrier()

            pltpu.sync_copy(x4_sh.at[pl.ds(r0, RT)], zbuf)
            pltpu.sync_copy(zeros16.at[pl.ds(r0, RT)], x4_sh.at[pl.ds(r0, RT)])

            def upd(i, c2):
                dv = d16b[i]
                xn = 0.9 * (dv * (zbuf[i] + dv * xbuf[i])) + h01b[i]
                xbuf[i] = xn
                rows[1024 + i] = dv * xn
                return c2
            lax.fori_loop(0, RT, upd, 0)
            pltpu.sync_copy(rows.at[pl.ds(1024, RT)], y_sh.at[pl.ds(r0, RT)])
            plsc.subcore_barrier()
            return c
        lax.fori_loop(0, K_HOPS, hop, 0)
        pltpu.sync_copy(xbuf, xfin.at[pl.ds(r0, RT)])


def _lsm_body(x_ref, o_ref):
    x = x_ref[...]
    m = jnp.max(x, axis=1, keepdims=True)
    ex = jnp.exp(x - m)
    lse = jnp.log(jnp.sum(ex, axis=1, keepdims=True)) + m
    o_ref[...] = x - lse


def _logsm(x):
    return pl.pallas_call(
        _lsm_body,
        out_shape=jax.ShapeDtypeStruct((NPAD, C_DIM), jnp.float32),
    )(x)


def _pad_idx(v, fill):
    pad = jnp.full((EPAD - E_EDGES,), fill, jnp.int32)
    return jnp.concatenate([v, pad]).reshape(NSUB, SUB)


def _pad_idx4(v, fill):
    pad = jnp.full((EPAD4 - 4 * E_EDGES,), fill, jnp.int32)
    return jnp.concatenate([v, pad]).reshape(NSUB4, SUB)


def kernel(attr_idx, edge_idx, n, d, W1, W2):
    del n, d
    four = jnp.arange(4, dtype=jnp.int32)
    ar4 = (attr_idx[0][:, None] * 4 + four).reshape(-1)
    ac4 = (attr_idx[1][:, None] * 4 + four).reshape(-1)
    ar2 = _pad_idx4(ar4, JUNK4)
    ac2 = _pad_idx4(ac4, 0)
    es2 = _pad_idx(edge_idx[0], 0)
    ed2 = _pad_idx(edge_idx[1], JUNK)
    w14 = jnp.concatenate(
        [W1, jnp.zeros((NPAD - D_FEAT, H_DIM), jnp.float32)]).reshape(
            NPAD4, C_DIM)
    zeros16 = jnp.zeros((NPAD, C_DIM), jnp.float32)
    ones = jnp.ones((8 * SUB, C_DIM), jnp.float32)

    xfin = _appnp_sc(ar2, ac2, es2, ed2, w14, W2, zeros16, ones)
    out = _logsm(xfin)
    return out[:N_NODES]


# speedup vs baseline: 1.3632x; 1.3632x over previous
import functools

import jax
import jax.numpy as jnp
from jax import lax
from jax.experimental import pallas as pl
from jax.experimental.pallas import tpu as pltpu
from jax.experimental.pallas import tpu_sc as plsc

N_NODES = 10000
NPAD = 10240
D_FEAT = 10000
H_DIM = 64
C_DIM = 16
E_EDGES = 320000
EPAD = 327680
SUB = 128
NSUB = EPAD // SUB
JUNK = N_NODES
K_HOPS = 10

NPAD4 = 4 * NPAD
EPAD4 = 4 * EPAD
NSUB4 = EPAD4 // SUB
JUNK4 = 4 * N_NODES

NC, NS = 2, 16
RT = NPAD // NS
RT4 = NPAD4 // NS
_MESH = plsc.VectorSubcoreMesh(core_axis_name="c", subcore_axis_name="s")


def _rsqrt_vec(x):
    i = plsc.bitcast(x, jnp.int32)
    i = jnp.int32(0x5F3759DF) - lax.shift_right_arithmetic(i, 1)
    y = plsc.bitcast(i, jnp.float32)
    for _ in range(3):
        y = y * (1.5 - 0.5 * x * y * y)
    return y


@functools.partial(
    pl.kernel,
    out_type=jax.ShapeDtypeStruct((NPAD, C_DIM), jnp.float32),
    mesh=_MESH,
    compiler_params=pltpu.CompilerParams(use_tc_tiling_on_sc=False, needs_layout_passes=False),
    scratch_types=(
        pltpu.VMEM_SHARED((NPAD4, C_DIM), jnp.float32),
        pltpu.VMEM_SHARED((NPAD, C_DIM), jnp.float32),
        pltpu.VMEM((16, SUB), jnp.int32),
        pltpu.VMEM((16, SUB), jnp.int32),
        pltpu.VMEM((16 * SUB, C_DIM), jnp.float32),
        pltpu.VMEM((H_DIM, C_DIM), jnp.float32),
        pltpu.VMEM((RT, C_DIM), jnp.float32),
        pltpu.VMEM((RT, C_DIM), jnp.float32),
        pltpu.VMEM((RT, C_DIM), jnp.float32),
        pltpu.VMEM((RT, C_DIM), jnp.float32),
        pltpu.SemaphoreType.DMA,
        pltpu.SemaphoreType.DMA,
    ),
)
def _appnp_sc(ar2, ac2, es2, ed2, w14, w2_hbm, zeros16, ones_hbm,
              xfin, x4_sh, y_sh, sbuf, dbuf, rows,
              w2buf, xbuf, h01b, d16b, zbuf, gsem, ssem):
    cid = lax.axis_index("c")
    sid = lax.axis_index("s")
    r4 = sid * RT4
    r0 = sid * RT

    def _drain(sem):
        pltpu.make_async_copy(zeros16.at[pl.ds(0, SUB)],
                              rows.at[pl.ds(0, SUB)], sem).wait()

    @pl.when(cid == 0)
    def _body():
        pltpu.sync_copy(zeros16.at[pl.ds(0, RT4)], x4_sh.at[pl.ds(r4, RT4)])
        pltpu.sync_copy(w2_hbm, w2buf)
        plsc.subcore_barrier()

        def sgrp(g, c2):
            p = lax.rem(g, 2) * 8
            gs = sid * (NSUB4 // NS) + g * 8
            pltpu.sync_copy(ac2.at[pl.ds(gs, 8)], sbuf.at[pl.ds(p, 8)])
            pltpu.sync_copy(ar2.at[pl.ds(gs, 8)], dbuf.at[pl.ds(p, 8)])
            for b in range(8):
                pltpu.async_copy(w14.at[sbuf.at[p + b]],
                                 rows.at[pl.ds((p + b) * SUB, SUB)], gsem)

            @pl.when(g > 0)
            def _dp():
                for b in range(8):
                    _drain(ssem)
            for b in range(8):
                _drain(gsem)
            for b in range(8):
                pltpu.async_copy(rows.at[pl.ds((p + b) * SUB, SUB)],
                                 x4_sh.at[dbuf.at[p + b]], ssem, add=True)
            return c2
        lax.fori_loop(0, NSUB4 // NS // 8, sgrp, 0)
        for b in range(8):
            _drain(ssem)

        plsc.subcore_barrier()

        def mm_chunk(ch, c2):
            pltpu.sync_copy(x4_sh.at[pl.ds(r4 + ch * 256, 256)],
                            rows.at[pl.ds(0, 256)])

            def mm_node(i, c3):
                node = ch * 64 + i
                acc = jnp.zeros((C_DIM,), jnp.float32)
                for q in range(4):
                    v = jnp.maximum(rows[4 * i + q], 0.0)
                    for j in range(C_DIM):
                        acc = acc + v[j] * w2buf[q * C_DIM + j]
                xbuf[node] = acc
                h01b[node] = 0.1 * acc
                return c3
            lax.fori_loop(0, 64, mm_node, 0)
            return c2
        lax.fori_loop(0, RT // 64, mm_chunk, 0)
        pltpu.sync_copy(zeros16.at[pl.ds(r0, RT)], x4_sh.at[pl.ds(r0, RT)])
        plsc.subcore_barrier()

        pltpu.sync_copy(ones_hbm, rows.at[pl.ds(0, 8 * SUB)])

        def dgrp(g, c2):
            gs = sid * (NSUB // NS) + g * 8
            pltpu.sync_copy(ed2.at[pl.ds(gs, 8)], sbuf.at[pl.ds(0, 8)])
            dls = [pltpu.async_copy(rows.at[pl.ds(b * SUB, SUB)],
                                    x4_sh.at[sbuf.at[b]], ssem, add=True)
                   for b in range(8)]
            for de in dls:
                de.wait()
            return c2
        lax.fori_loop(0, NSUB // NS // 8, dgrp, 0)
        plsc.subcore_barrier()

        pltpu.sync_copy(x4_sh.at[pl.ds(r0, RT)], zbuf)
        pltpu.sync_copy(zeros16.at[pl.ds(r0, RT)], x4_sh.at[pl.ds(r0, RT)])

        def norm(i, c2):
            dv = _rsqrt_vec(zbuf[i] + 1.0)
            d16b[i] = dv
            rows[1024 + i] = dv * xbuf[i]
            return c2
        lax.fori_loop(0, RT, norm, 0)
        pltpu.sync_copy(rows.at[pl.ds(1024, RT)], y_sh.at[pl.ds(r0, RT)])
        plsc.subcore_barrier()

        def hop(k, c):
            def grp(g, c2):
                p = lax.rem(g, 2) * 8
                gs = sid * (NSUB // NS) + g * 8
                pltpu.sync_copy(es2.at[pl.ds(gs, 8)], sbuf.at[pl.ds(p, 8)])
                pltpu.sync_copy(ed2.at[pl.ds(gs, 8)], dbuf.at[pl.ds(p, 8)])
                for b in range(8):
                    pltpu.async_copy(y_sh.at[sbuf.at[p + b]],
                                     rows.at[pl.ds((p + b) * SUB, SUB)], gsem)

                @pl.when(g > 0)
                def _dp():
                    for b in range(8):
                        _drain(ssem)
                for b in range(8):
                    _drain(gsem)
                for b in range(8):
                    pltpu.async_copy(rows.at[pl.ds((p + b) * SUB, SUB)],
                                     x4_sh.at[dbuf.at[p + b]], ssem, add=True)
                return c2
            lax.fori_loop(0, NSUB // NS // 8, grp, 0)
            for b in range(8):
                _drain(ssem)
            plsc.subcore_barrier()

            pltpu.sync_copy(x4_sh.at[pl.ds(r0, RT)], zbuf)
            pltpu.sync_copy(zeros16.at[pl.ds(r0, RT)], x4_sh.at[pl.ds(r0, RT)])

            def upd(i, c2):
                dv = d16b[i]
                xn = 0.9 * (dv * (zbuf[i] + dv * xbuf[i])) + h01b[i]
                xbuf[i] = xn
                rows[1024 + i] = dv * xn
                return c2
            lax.fori_loop(0, RT, upd, 0)
            pltpu.sync_copy(rows.at[pl.ds(1024, RT)], y_sh.at[pl.ds(r0, RT)])
            plsc.subcore_barrier()
            return c
        lax.fori_loop(0, K_HOPS, hop, 0)
        pltpu.sync_copy(xbuf, xfin.at[pl.ds(r0, RT)])


def _prep_body(attr_ref, edge_ref, aro, aco, eso, edo):
    nr = E_EDGES // SUB

    def expand(a, fill, out_ref):
        a4 = a.reshape(nr, SUB) * 4
        for qq in range(4):
            out_ref[qq * nr:(qq + 1) * nr] = a4 + qq
        out_ref[4 * nr:] = jnp.full((NSUB4 - 4 * nr, SUB), fill, jnp.int32)

    expand(attr_ref[0], JUNK4, aro)
    expand(attr_ref[1], 0, aco)

    def padded(a, fill, out_ref):
        out_ref[0:E_EDGES // SUB] = a.reshape(E_EDGES // SUB, SUB)
        out_ref[E_EDGES // SUB:] = jnp.full((NSUB - E_EDGES // SUB, SUB),
                                            fill, jnp.int32)

    padded(edge_ref[0], 0, eso)
    padded(edge_ref[1], JUNK, edo)


def _prep(attr_idx, edge_idx):
    return pl.pallas_call(
        _prep_body,
        out_shape=(jax.ShapeDtypeStruct((NSUB4, SUB), jnp.int32),
                   jax.ShapeDtypeStruct((NSUB4, SUB), jnp.int32),
                   jax.ShapeDtypeStruct((NSUB, SUB), jnp.int32),
                   jax.ShapeDtypeStruct((NSUB, SUB), jnp.int32)),
    )(attr_idx, edge_idx)


def _lsm_body(x_ref, o_ref):
    x = x_ref[...]
    m = jnp.max(x, axis=1, keepdims=True)
    ex = jnp.exp(x - m)
    lse = jnp.log(jnp.sum(ex, axis=1, keepdims=True)) + m
    o_ref[...] = x - lse


def _logsm(x):
    return pl.pallas_call(
        _lsm_body,
        out_shape=jax.ShapeDtypeStruct((NPAD, C_DIM), jnp.float32),
    )(x)


def _pad_idx(v, fill):
    pad = jnp.full((EPAD - E_EDGES,), fill, jnp.int32)
    return jnp.concatenate([v, pad]).reshape(NSUB, SUB)


def _pad_idx4(v, fill):
    pad = jnp.full((EPAD4 - 4 * E_EDGES,), fill, jnp.int32)
    return jnp.concatenate([v, pad]).reshape(NSUB4, SUB)


def kernel(attr_idx, edge_idx, n, d, W1, W2):
    del n, d
    ar2, ac2, es2, ed2 = _prep(attr_idx, edge_idx)
    w14 = W1.reshape(4 * D_FEAT, C_DIM)
    zeros16 = jnp.zeros((NPAD, C_DIM), jnp.float32)
    ones = jnp.ones((8 * SUB, C_DIM), jnp.float32)

    xfin = _appnp_sc(ar2, ac2, es2, ed2, w14, W2, zeros16, ones)
    out = _logsm(xfin)
    return out[:N_NODES]


# speedup vs baseline: 1.7058x; 1.2513x over previous
import functools

import jax
import jax.numpy as jnp
from jax import lax
from jax.experimental import pallas as pl
from jax.experimental.pallas import tpu as pltpu
from jax.experimental.pallas import tpu_sc as plsc

N_NODES = 10000
NPAD = 10240
D_FEAT = 10000
H_DIM = 64
C_DIM = 16
E_EDGES = 320000
EPAD = 327680
SUB = 128
NSUB = EPAD // SUB
JUNK = N_NODES
K_HOPS = 10

NPAD4 = 4 * NPAD
EPAD4 = 4 * EPAD
NSUB4 = EPAD4 // SUB
JUNK4 = 4 * N_NODES

NC, NS = 2, 16
RT = NPAD // NS
RT4 = NPAD4 // NS
_MESH = plsc.VectorSubcoreMesh(core_axis_name="c", subcore_axis_name="s")


def _rsqrt_vec(x):
    i = plsc.bitcast(x, jnp.int32)
    i = jnp.int32(0x5F3759DF) - lax.shift_right_arithmetic(i, 1)
    y = plsc.bitcast(i, jnp.float32)
    for _ in range(3):
        y = y * (1.5 - 0.5 * x * y * y)
    return y


@functools.partial(
    pl.kernel,
    out_type=jax.ShapeDtypeStruct((NC, NPAD4, C_DIM), jnp.float32),
    mesh=_MESH,
    compiler_params=pltpu.CompilerParams(use_tc_tiling_on_sc=False,
                                         needs_layout_passes=False),
    scratch_types=(
        pltpu.VMEM_SHARED((NPAD4, C_DIM), jnp.float32),
        pltpu.VMEM((16, SUB), jnp.int32),
        pltpu.VMEM((16, SUB), jnp.int32),
        pltpu.VMEM((16 * SUB, C_DIM), jnp.float32),
        pltpu.SemaphoreType.DMA,
        pltpu.SemaphoreType.DMA,
        pltpu.SemaphoreType.DMA,
    ),
)
def _spmm2(ar2, ac2, w14, zeros16, xp_out,
           x4_sh, sbuf, dbuf, rows, gsem, ssem, isem):
    cid = lax.axis_index("c")
    sid = lax.axis_index("s")
    wid = sid * NC + cid

    def _drain(sem):
        pltpu.make_async_copy(zeros16.at[pl.ds(0, SUB)],
                              rows.at[pl.ds(0, SUB)], sem).wait()

    def _drain_idx():
        pltpu.make_async_copy(ar2.at[pl.ds(0, 8)], sbuf.at[pl.ds(0, 8)],
                              isem).wait()

    rz = sid * RT4
    for t in range(4):
        pltpu.sync_copy(zeros16.at[pl.ds(0, RT4 // 4)],
                        x4_sh.at[pl.ds(rz + t * (RT4 // 4), RT4 // 4)])
    plsc.subcore_barrier()

    ng = NSUB4 // (NC * NS) // 8
    s0 = wid * (NSUB4 // (NC * NS))
    pltpu.async_copy(ac2.at[pl.ds(s0, 8)], sbuf.at[pl.ds(0, 8)], isem)
    pltpu.async_copy(ar2.at[pl.ds(s0, 8)], dbuf.at[pl.ds(0, 8)], isem)

    def sgrp(g, c2):
        p = lax.rem(g, 2) * 8
        gs = s0 + g * 8
        _drain_idx()
        _drain_idx()
        for b in range(8):
            pltpu.async_copy(w14.at[sbuf.at[p + b]],
                             rows.at[pl.ds((p + b) * SUB, SUB)], gsem)

        @pl.when(g > 0)
        def _dp():
            for b in range(8):
                _drain(ssem)

        @pl.when(g < ng - 1)
        def _pf():
            pltpu.async_copy(ac2.at[pl.ds(gs + 8, 8)],
                             sbuf.at[pl.ds(8 - p, 8)], isem)
            pltpu.async_copy(ar2.at[pl.ds(gs + 8, 8)],
                             dbuf.at[pl.ds(8 - p, 8)], isem)
        for b in range(8):
            _drain(gsem)
        for b in range(8):
            pltpu.async_copy(rows.at[pl.ds((p + b) * SUB, SUB)],
                             x4_sh.at[dbuf.at[p + b]], ssem, add=True)
        return c2
    lax.fori_loop(0, ng, sgrp, 0)
    for b in range(8):
        _drain(ssem)
    plsc.subcore_barrier()
    pltpu.sync_copy(x4_sh.at[pl.ds(rz, RT4)],
                    xp_out.at[cid, pl.ds(rz, RT4)])


@functools.partial(
    pl.kernel,
    out_type=jax.ShapeDtypeStruct((NPAD, C_DIM), jnp.float32),
    mesh=_MESH,
    compiler_params=pltpu.CompilerParams(use_tc_tiling_on_sc=False, needs_layout_passes=False),
    scratch_types=(
        pltpu.VMEM_SHARED((NPAD, C_DIM), jnp.float32),
        pltpu.VMEM_SHARED((NPAD, C_DIM), jnp.float32),
        pltpu.VMEM((16, SUB), jnp.int32),
        pltpu.VMEM((16, SUB), jnp.int32),
        pltpu.VMEM((16 * SUB, C_DIM), jnp.float32),
        pltpu.VMEM((H_DIM, C_DIM), jnp.float32),
        pltpu.VMEM((RT, C_DIM), jnp.float32),
        pltpu.VMEM((RT, C_DIM), jnp.float32),
        pltpu.VMEM((RT, C_DIM), jnp.float32),
        pltpu.VMEM((RT, C_DIM), jnp.float32),
        pltpu.SemaphoreType.DMA,
        pltpu.SemaphoreType.DMA,
        pltpu.SemaphoreType.DMA,
    ),
)
def _appnp_sc(xp, es2, ed2, w2_hbm, zeros16, ones_hbm,
              xfin, z_sh, y_sh, sbuf, dbuf, rows,
              w2buf, xbuf, h01b, d16b, zbuf, gsem, ssem, isem):
    cid = lax.axis_index("c")
    sid = lax.axis_index("s")
    r4 = sid * RT4
    r0 = sid * RT

    def _drain(sem):
        pltpu.make_async_copy(zeros16.at[pl.ds(0, SUB)],
                              rows.at[pl.ds(0, SUB)], sem).wait()

    def _drain_idx():
        pltpu.make_async_copy(es2.at[pl.ds(0, 8)], sbuf.at[pl.ds(0, 8)],
                              isem).wait()

    @pl.when(cid == 0)
    def _body():
        pltpu.sync_copy(zeros16.at[pl.ds(r0, RT)], z_sh.at[pl.ds(r0, RT)])
        pltpu.sync_copy(w2_hbm, w2buf)

        def mm_chunk(ch, c2):
            pltpu.sync_copy(xp.at[0, pl.ds(r4 + ch * 256, 256)],
                            rows.at[pl.ds(0, 256)])
            pltpu.sync_copy(xp.at[1, pl.ds(r4 + ch * 256, 256)],
                            rows.at[pl.ds(256, 256)])

            def mm_node(i, c3):
                node = ch * 64 + i
                acc = jnp.zeros((C_DIM,), jnp.float32)
                for q in range(4):
                    v = jnp.maximum(rows[4 * i + q] + rows[256 + 4 * i + q],
                                    0.0)
                    for j in range(C_DIM):
                        acc = acc + v[j] * w2buf[q * C_DIM + j]
                xbuf[node] = acc
                h01b[node] = 0.1 * acc
                return c3
            lax.fori_loop(0, 64, mm_node, 0)
            return c2
        lax.fori_loop(0, RT // 64, mm_chunk, 0)
        plsc.subcore_barrier()

        pltpu.sync_copy(ones_hbm, rows.at[pl.ds(0, 8 * SUB)])

        def dgrp(g, c2):
            gs = sid * (NSUB // NS) + g * 8
            pltpu.sync_copy(ed2.at[pl.ds(gs, 8)], sbuf.at[pl.ds(0, 8)])
            dls = [pltpu.async_copy(rows.at[pl.ds(b * SUB, SUB)],
                                    z_sh.at[sbuf.at[b]], ssem, add=True)
                   for b in range(8)]
            for de in dls:
                de.wait()
            return c2
        lax.fori_loop(0, NSUB // NS // 8, dgrp, 0)
        plsc.subcore_barrier()

        pltpu.sync_copy(z_sh.at[pl.ds(r0, RT)], zbuf)
        pltpu.sync_copy(zeros16.at[pl.ds(r0, RT)], z_sh.at[pl.ds(r0, RT)])

        def norm(i, c2):
            dv = _rsqrt_vec(zbuf[i] + 1.0)
            d16b[i] = dv
            rows[1024 + i] = dv * xbuf[i]
            return c2
        lax.fori_loop(0, RT, norm, 0)
        pltpu.sync_copy(rows.at[pl.ds(1024, RT)], y_sh.at[pl.ds(r0, RT)])
        plsc.subcore_barrier()

        def hop(k, c):
            nge = NSUB // NS // 8
            e0 = sid * (NSUB // NS)
            pltpu.async_copy(es2.at[pl.ds(e0, 8)], sbuf.at[pl.ds(0, 8)], isem)
            pltpu.async_copy(ed2.at[pl.ds(e0, 8)], dbuf.at[pl.ds(0, 8)], isem)

            def grp(g, c2):
                p = lax.rem(g, 2) * 8
                gs = e0 + g * 8
                _drain_idx()
                _drain_idx()
                for b in range(8):
                    pltpu.async_copy(y_sh.at[sbuf.at[p + b]],
                                     rows.at[pl.ds((p + b) * SUB, SUB)], gsem)

                @pl.when(g > 0)
                def _dp():
                    for b in range(8):
                        _drain(ssem)

                @pl.when(g < nge - 1)
                def _pf():
                    pltpu.async_copy(es2.at[pl.ds(gs + 8, 8)],
                                     sbuf.at[pl.ds(8 - p, 8)], isem)
                    pltpu.async_copy(ed2.at[pl.ds(gs + 8, 8)],
                                     dbuf.at[pl.ds(8 - p, 8)], isem)
                for b in range(8):
                    _drain(gsem)
                for b in range(8):
                    pltpu.async_copy(rows.at[pl.ds((p + b) * SUB, SUB)],
                                     z_sh.at[dbuf.at[p + b]], ssem, add=True)
                return c2
            lax.fori_loop(0, nge, grp, 0)
            for b in range(8):
                _drain(ssem)
            plsc.subcore_barrier()

            pltpu.sync_copy(z_sh.at[pl.ds(r0, RT)], zbuf)
            pltpu.sync_copy(zeros16.at[pl.ds(r0, RT)], z_sh.at[pl.ds(r0, RT)])

            def upd(i, c2):
                dv = d16b[i]
                xn = 0.9 * (dv * (zbuf[i] + dv * xbuf[i])) + h01b[i]
                xbuf[i] = xn
                rows[1024 + i] = dv * xn
                return c2
            lax.fori_loop(0, RT, upd, 0)
            pltpu.sync_copy(rows.at[pl.ds(1024, RT)], y_sh.at[pl.ds(r0, RT)])
            plsc.subcore_barrier()
            return c
        lax.fori_loop(0, K_HOPS, hop, 0)
        pltpu.sync_copy(xbuf, xfin.at[pl.ds(r0, RT)])


def _prep_body(attr_ref, edge_ref, aro, aco, eso, edo):
    nr = E_EDGES // SUB

    def expand(a, fill, out_ref):
        a4 = a.reshape(nr, SUB) * 4
        for qq in range(4):
            out_ref[qq * nr:(qq + 1) * nr] = a4 + qq
        out_ref[4 * nr:] = jnp.full((NSUB4 - 4 * nr, SUB), fill, jnp.int32)

    expand(attr_ref[0], JUNK4, aro)
    expand(attr_ref[1], 0, aco)

    def padded(a, fill, out_ref):
        out_ref[0:E_EDGES // SUB] = a.reshape(E_EDGES // SUB, SUB)
        out_ref[E_EDGES // SUB:] = jnp.full((NSUB - E_EDGES // SUB, SUB),
                                            fill, jnp.int32)

    padded(edge_ref[0], 0, eso)
    padded(edge_ref[1], JUNK, edo)


def _prep(attr_idx, edge_idx):
    return pl.pallas_call(
        _prep_body,
        out_shape=(jax.ShapeDtypeStruct((NSUB4, SUB), jnp.int32),
                   jax.ShapeDtypeStruct((NSUB4, SUB), jnp.int32),
                   jax.ShapeDtypeStruct((NSUB, SUB), jnp.int32),
                   jax.ShapeDtypeStruct((NSUB, SUB), jnp.int32)),
    )(attr_idx, edge_idx)


def _lsm_body(x_ref, o_ref):
    x = x_ref[...]
    m = jnp.max(x, axis=1, keepdims=True)
    ex = jnp.exp(x - m)
    lse = jnp.log(jnp.sum(ex, axis=1, keepdims=True)) + m
    o_ref[...] = x - lse


def _logsm(x):
    return pl.pallas_call(
        _lsm_body,
        out_shape=jax.ShapeDtypeStruct((NPAD, C_DIM), jnp.float32),
    )(x)


def _pad_idx(v, fill):
    pad = jnp.full((EPAD - E_EDGES,), fill, jnp.int32)
    return jnp.concatenate([v, pad]).reshape(NSUB, SUB)


def _pad_idx4(v, fill):
    pad = jnp.full((EPAD4 - 4 * E_EDGES,), fill, jnp.int32)
    return jnp.concatenate([v, pad]).reshape(NSUB4, SUB)


def kernel(attr_idx, edge_idx, n, d, W1, W2):
    del n, d
    ar2, ac2, es2, ed2 = _prep(attr_idx, edge_idx)
    w14 = W1.reshape(4 * D_FEAT, C_DIM)
    zeros16 = jnp.zeros((NPAD, C_DIM), jnp.float32)
    ones = jnp.ones((8 * SUB, C_DIM), jnp.float32)

    xp = _spmm2(ar2, ac2, w14, zeros16)
    xfin = _appnp_sc(xp, es2, ed2, W2, zeros16, ones)
    out = _logsm(xfin)
    return out[:N_NODES]


# speedup vs baseline: 1.7531x; 1.0278x over previous
import functools

import jax
import jax.numpy as jnp
from jax import lax
from jax.experimental import pallas as pl
from jax.experimental.pallas import tpu as pltpu
from jax.experimental.pallas import tpu_sc as plsc

N_NODES = 10000
NPAD = 10240
D_FEAT = 10000
H_DIM = 64
C_DIM = 16
E_EDGES = 320000
EPAD = 327680
SUB = 128
NSUB = EPAD // SUB
JUNK = N_NODES
K_HOPS = 10

NPAD4 = 4 * NPAD
EPAD4 = 4 * EPAD
NSUB4 = EPAD4 // SUB
JUNK4 = 4 * N_NODES

NC, NS = 2, 16
RT = NPAD // NS
RT4 = NPAD4 // NS
_MESH = plsc.VectorSubcoreMesh(core_axis_name="c", subcore_axis_name="s")


def _rsqrt_vec(x):
    i = plsc.bitcast(x, jnp.int32)
    i = jnp.int32(0x5F3759DF) - lax.shift_right_arithmetic(i, 1)
    y = plsc.bitcast(i, jnp.float32)
    for _ in range(3):
        y = y * (1.5 - 0.5 * x * y * y)
    return y


@functools.partial(
    pl.kernel,
    out_type=jax.ShapeDtypeStruct((NC, NPAD4, C_DIM), jnp.float32),
    mesh=_MESH,
    compiler_params=pltpu.CompilerParams(use_tc_tiling_on_sc=False,
                                         needs_layout_passes=False),
    scratch_types=(
        pltpu.VMEM_SHARED((NPAD4, C_DIM), jnp.float32),
        pltpu.VMEM((16, SUB), jnp.int32),
        pltpu.VMEM((16, SUB), jnp.int32),
        pltpu.VMEM((16 * SUB, C_DIM), jnp.float32),
        pltpu.SemaphoreType.DMA,
        pltpu.SemaphoreType.DMA,
        pltpu.SemaphoreType.DMA,
    ),
)
def _spmm2(ar2, ac2, w14, zeros16, xp_out,
           x4_sh, sbuf, dbuf, rows, gsem, ssem, isem):
    cid = lax.axis_index("c")
    sid = lax.axis_index("s")
    wid = sid * NC + cid

    def _drain(sem):
        pltpu.make_async_copy(zeros16.at[pl.ds(0, SUB)],
                              rows.at[pl.ds(0, SUB)], sem).wait()

    def _drain_idx():
        pltpu.make_async_copy(ar2.at[pl.ds(0, 8)], sbuf.at[pl.ds(0, 8)],
                              isem).wait()

    rz = sid * RT4
    for t in range(4):
        pltpu.sync_copy(zeros16.at[pl.ds(0, RT4 // 4)],
                        x4_sh.at[pl.ds(rz + t * (RT4 // 4), RT4 // 4)])
    plsc.subcore_barrier()

    ng = NSUB4 // (NC * NS) // 8
    s0 = wid * (NSUB4 // (NC * NS))
    pltpu.async_copy(ac2.at[pl.ds(s0, 8)], sbuf.at[pl.ds(0, 8)], isem)
    pltpu.async_copy(ar2.at[pl.ds(s0, 8)], dbuf.at[pl.ds(0, 8)], isem)

    def sgrp(g, c2):
        p = lax.rem(g, 2) * 8
        gs = s0 + g * 8
        _drain_idx()
        _drain_idx()
        for b in range(8):
            pltpu.async_copy(w14.at[sbuf.at[p + b]],
                             rows.at[pl.ds((p + b) * SUB, SUB)], gsem)

        @pl.when(g > 0)
        def _dp():
            for b in range(8):
                _drain(ssem)

        @pl.when(g < ng - 1)
        def _pf():
            pltpu.async_copy(ac2.at[pl.ds(gs + 8, 8)],
                             sbuf.at[pl.ds(8 - p, 8)], isem)
            pltpu.async_copy(ar2.at[pl.ds(gs + 8, 8)],
                             dbuf.at[pl.ds(8 - p, 8)], isem)
        for b in range(8):
            _drain(gsem)
        for b in range(8):
            pltpu.async_copy(rows.at[pl.ds((p + b) * SUB, SUB)],
                             x4_sh.at[dbuf.at[p + b]], ssem, add=True)
        return c2
    lax.fori_loop(0, ng, sgrp, 0)
    for b in range(8):
        _drain(ssem)
    plsc.subcore_barrier()
    pltpu.sync_copy(x4_sh.at[pl.ds(rz, RT4)],
                    xp_out.at[cid, pl.ds(rz, RT4)])


@functools.partial(
    pl.kernel,
    out_type=jax.ShapeDtypeStruct((NPAD, C_DIM), jnp.float32),
    mesh=_MESH,
    compiler_params=pltpu.CompilerParams(use_tc_tiling_on_sc=False, needs_layout_passes=False),
    scratch_types=(
        pltpu.VMEM_SHARED((NPAD, C_DIM), jnp.float32),
        pltpu.VMEM_SHARED((NPAD, C_DIM), jnp.float32),
        pltpu.VMEM((32, SUB), jnp.int32),
        pltpu.VMEM((32, SUB), jnp.int32),
        pltpu.VMEM((32 * SUB, C_DIM), jnp.float32),
        pltpu.VMEM((H_DIM, C_DIM), jnp.float32),
        pltpu.VMEM((RT, C_DIM), jnp.float32),
        pltpu.VMEM((RT, C_DIM), jnp.float32),
        pltpu.VMEM((RT, C_DIM), jnp.float32),
        pltpu.SemaphoreType.DMA,
        pltpu.SemaphoreType.DMA,
        pltpu.SemaphoreType.DMA,
        pltpu.SemaphoreType.DMA,
    ),
)
def _appnp_sc(xp, es2, ed2, w2_hbm, zeros16, ones_hbm,
              xfin, z_sh, y_sh, sbuf, dbuf, rows,
              w2buf, xbuf, h01b, d16b, gsem, ssem, isem, gsem2):
    cid = lax.axis_index("c")
    sid = lax.axis_index("s")
    r4 = sid * RT4
    r0 = sid * RT

    def _drain(sem):
        pltpu.make_async_copy(zeros16.at[pl.ds(0, SUB)],
                              rows.at[pl.ds(0, SUB)], sem).wait()

    def _drain_idx16():
        pltpu.make_async_copy(es2.at[pl.ds(0, 16)], sbuf.at[pl.ds(0, 16)],
                              isem).wait()

    @pl.when(cid == 0)
    def _body():
        pltpu.sync_copy(zeros16.at[pl.ds(r0, RT)], z_sh.at[pl.ds(r0, RT)])
        pltpu.sync_copy(w2_hbm, w2buf)

        def mm_chunk(ch, c2):
            pltpu.sync_copy(xp.at[0, pl.ds(r4 + ch * 256, 256)],
                            rows.at[pl.ds(0, 256)])
            pltpu.sync_copy(xp.at[1, pl.ds(r4 + ch * 256, 256)],
                            rows.at[pl.ds(256, 256)])

            def mm_node(i, c3):
                node = ch * 64 + i
                acc = jnp.zeros((C_DIM,), jnp.float32)
                for q in range(4):
                    v = jnp.maximum(rows[4 * i + q] + rows[256 + 4 * i + q],
                                    0.0)
                    for j in range(C_DIM):
                        acc = acc + v[j] * w2buf[q * C_DIM + j]
                xbuf[node] = acc
                h01b[node] = 0.1 * acc
                return c3
            lax.fori_loop(0, 64, mm_node, 0)
            return c2
        lax.fori_loop(0, RT // 64, mm_chunk, 0)
        plsc.subcore_barrier()

        pltpu.sync_copy(ones_hbm, rows.at[pl.ds(0, 8 * SUB)])

        def dgrp(g, c2):
            gs = sid * (NSUB // NS) + g * 8
            pltpu.sync_copy(ed2.at[pl.ds(gs, 8)], sbuf.at[pl.ds(0, 8)])
            dls = [pltpu.async_copy(rows.at[pl.ds(b * SUB, SUB)],
                                    z_sh.at[sbuf.at[b]], ssem, add=True)
                   for b in range(8)]
            for de in dls:
                de.wait()
            return c2
        lax.fori_loop(0, NSUB // NS // 8, dgrp, 0)
        plsc.subcore_barrier()

        pltpu.sync_copy(z_sh.at[pl.ds(r0, RT)], rows.at[pl.ds(0, RT)])
        pltpu.sync_copy(zeros16.at[pl.ds(r0, RT)], z_sh.at[pl.ds(r0, RT)])

        def norm(i, c2):
            dv = _rsqrt_vec(rows[i] + 1.0)
            d16b[i] = dv
            rows[1024 + i] = dv * xbuf[i]
            return c2
        lax.fori_loop(0, RT, norm, 0)
        pltpu.sync_copy(rows.at[pl.ds(1024, RT)], y_sh.at[pl.ds(r0, RT)])
        plsc.subcore_barrier()

        def hop(k, c):
            npair = NSUB // NS // 16
            e0 = sid * (NSUB // NS)
            pltpu.async_copy(es2.at[pl.ds(e0, 16)], sbuf.at[pl.ds(0, 16)],
                             isem)
            pltpu.async_copy(ed2.at[pl.ds(e0, 16)], dbuf.at[pl.ds(0, 16)],
                             isem)

            def pair(it, c2):
                q = lax.rem(it, 2) * 16
                gs = e0 + it * 16
                _drain_idx16()
                _drain_idx16()
                for b in range(8):
                    pltpu.async_copy(y_sh.at[sbuf.at[q + b]],
                                     rows.at[pl.ds((q + b) * SUB, SUB)], gsem)
                for b in range(8):
                    pltpu.async_copy(y_sh.at[sbuf.at[q + 8 + b]],
                                     rows.at[pl.ds((q + 8 + b) * SUB, SUB)],
                                     gsem2)

                @pl.when(it > 0)
                def _dp():
                    for b in range(16):
                        _drain(ssem)

                @pl.when(it < npair - 1)
                def _pf():
                    pltpu.async_copy(es2.at[pl.ds(gs + 16, 16)],
                                     sbuf.at[pl.ds(16 - q, 16)], isem)
                    pltpu.async_copy(ed2.at[pl.ds(gs + 16, 16)],
                                     dbuf.at[pl.ds(16 - q, 16)], isem)
                for b in range(8):
                    _drain(gsem)
                for b in range(8):
                    pltpu.async_copy(rows.at[pl.ds((q + b) * SUB, SUB)],
                                     z_sh.at[dbuf.at[q + b]], ssem, add=True)
                for b in range(8):
                    _drain(gsem2)
                for b in range(8):
                    pltpu.async_copy(rows.at[pl.ds((q + 8 + b) * SUB, SUB)],
                                     z_sh.at[dbuf.at[q + 8 + b]], ssem,
                                     add=True)
                return c2
            lax.fori_loop(0, npair, pair, 0)
            for b in range(16):
                _drain(ssem)
            plsc.subcore_barrier()

            pltpu.sync_copy(z_sh.at[pl.ds(r0, RT)], rows.at[pl.ds(0, RT)])
            pltpu.sync_copy(zeros16.at[pl.ds(r0, RT)], z_sh.at[pl.ds(r0, RT)])

            def upd(i, c2):
                dv = d16b[i]
                xn = 0.9 * (dv * (rows[i] + dv * xbuf[i])) + h01b[i]
                xbuf[i] = xn
                rows[1024 + i] = dv * xn
                return c2
            lax.fori_loop(0, RT, upd, 0)
            pltpu.sync_copy(rows.at[pl.ds(1024, RT)], y_sh.at[pl.ds(r0, RT)])
            plsc.subcore_barrier()
            return c
        lax.fori_loop(0, K_HOPS, hop, 0)
        pltpu.sync_copy(xbuf, xfin.at[pl.ds(r0, RT)])


def _prep_body(attr_ref, edge_ref, aro, aco, eso, edo):
    nr = E_EDGES // SUB

    def expand(a, fill, out_ref):
        a4 = a.reshape(nr, SUB) * 4
        for qq in range(4):
            out_ref[qq * nr:(qq + 1) * nr] = a4 + qq
        out_ref[4 * nr:] = jnp.full((NSUB4 - 4 * nr, SUB), fill, jnp.int32)

    expand(attr_ref[0], JUNK4, aro)
    expand(attr_ref[1], 0, aco)

    def padded(a, fill, out_ref):
        out_ref[0:E_EDGES // SUB] = a.reshape(E_EDGES // SUB, SUB)
        out_ref[E_EDGES // SUB:] = jnp.full((NSUB - E_EDGES // SUB, SUB),
                                            fill, jnp.int32)

    padded(edge_ref[0], 0, eso)
    padded(edge_ref[1], JUNK, edo)


def _prep(attr_idx, edge_idx):
    return pl.pallas_call(
        _prep_body,
        out_shape=(jax.ShapeDtypeStruct((NSUB4, SUB), jnp.int32),
                   jax.ShapeDtypeStruct((NSUB4, SUB), jnp.int32),
                   jax.ShapeDtypeStruct((NSUB, SUB), jnp.int32),
                   jax.ShapeDtypeStruct((NSUB, SUB), jnp.int32)),
    )(attr_idx, edge_idx)


def _lsm_body(x_ref, o_ref):
    x = x_ref[...]
    m = jnp.max(x, axis=1, keepdims=True)
    ex = jnp.exp(x - m)
    lse = jnp.log(jnp.sum(ex, axis=1, keepdims=True)) + m
    o_ref[...] = x - lse


def _logsm(x):
    return pl.pallas_call(
        _lsm_body,
        out_shape=jax.ShapeDtypeStruct((NPAD, C_DIM), jnp.float32),
    )(x)


def _pad_idx(v, fill):
    pad = jnp.full((EPAD - E_EDGES,), fill, jnp.int32)
    return jnp.concatenate([v, pad]).reshape(NSUB, SUB)


def _pad_idx4(v, fill):
    pad = jnp.full((EPAD4 - 4 * E_EDGES,), fill, jnp.int32)
    return jnp.concatenate([v, pad]).reshape(NSUB4, SUB)


def kernel(attr_idx, edge_idx, n, d, W1, W2):
    del n, d
    ar2, ac2, es2, ed2 = _prep(attr_idx, edge_idx)
    w14 = W1.reshape(4 * D_FEAT, C_DIM)
    zeros16 = jnp.zeros((NPAD, C_DIM), jnp.float32)
    ones = jnp.ones((8 * SUB, C_DIM), jnp.float32)

    xp = _spmm2(ar2, ac2, w14, zeros16)
    xfin = _appnp_sc(xp, es2, ed2, W2, zeros16, ones)
    out = _logsm(xfin)
    return out[:N_NODES]
